# baseline (speedup 1.0000x reference)
import functools

import jax
import jax.numpy as jnp
from jax import lax
from jax.experimental import pallas as pl
from jax.experimental.pallas import tpu as pltpu
from jax.experimental.pallas import tpu_sc as plsc

_B = 4096
_S = 200
_D = 32
_V = 1000000
_NW = 32
_TC = 800
_NCH = _V // _TC
_KMAX = 40


def _tr_body(ttT, out, in0, in1, out0, out1, rs0, rs1, ws0, ws1):
    cid = lax.axis_index("c")
    sid = lax.axis_index("s")
    wid = sid * 2 + cid

    ins = (in0, in1)
    outs = (out0, out1)
    rsems = (rs0, rs1)
    wsems = (ws0, ws1)

    iota = lax.iota(jnp.int32, 16)
    iota_hi = iota + 16

    def chunk_id(k):
        return wid + _NW * k

    def fire_read(k, slot):
        c = chunk_id(k)

        @pl.when(c < _NCH)
        def _():
            pltpu.async_copy(
                ttT.at[:, pl.ds(c * _TC, _TC)], ins[slot], rsems[slot]
            )

    def drain(ref_dst, sem, src):
        pltpu.make_async_copy(src, ref_dst, sem).wait()

    fire_read(0, 0)
    for k in range(_KMAX):
        slot = k % 2
        c = chunk_id(k)
        if k + 1 < _KMAX:
            fire_read(k + 1, (k + 1) % 2)

        @pl.when(c < _NCH)
        def _(k=k, slot=slot, c=c):
            drain(ins[slot], rsems[slot], ttT.at[:, pl.ds(0, _TC)])
            if k >= 2:
                drain(out.at[pl.ds(0, _TC)], wsems[slot], outs[slot])

            @pl.loop(0, _TC)
            def _t(t):
                ts = jnp.full((16,), t, jnp.int32)
                v0 = plsc.load_gather(ins[slot], [iota, ts])
                v1 = plsc.load_gather(ins[slot], [iota_hi, ts])
                outs[slot][t, pl.ds(0, 16)] = v0
                outs[slot][t, pl.ds(16, 16)] = v1

            pltpu.async_copy(
                outs[slot], out.at[pl.ds(c * _TC, _TC)], wsems[slot]
            )

    for k in (_KMAX - 2, _KMAX - 1):
        slot = k % 2

        @pl.when(chunk_id(k) < _NCH)
        def _(slot=slot):
            drain(out.at[pl.ds(0, _TC)], wsems[slot], outs[slot])


def _lk_body(
    xv, tab, pos_hbm, out,
    xidx, pos_v,
    r0, r1, r2, r3, t0, t1, t2, t3,
    g0, g1, g2, g3, w0, w1, w2, w3,
):
    cid = lax.axis_index("c")
    sid = lax.axis_index("s")
    wid = sid * 2 + cid

    rows = (r0, r1, r2, r3)
    tiles = (t0, t1, t2, t3)
    gsems = (g0, g1, g2, g3)
    wsems = (w0, w1, w2, w3)

    pltpu.sync_copy(pos_hbm, pos_v)
    pltpu.sync_copy(xv.at[:, wid], xidx)

    iota = lax.iota(jnp.int32, 16)
    bcs = [iota + 16 * b for b in range(8)]

    def fire_gather(s, j):
        pltpu.async_copy(
            tab.at[xidx.at[s // 8, lax.rem(s, 8)]], rows[j], gsems[j]
        )

    def drain(dst, sem, src):
        pltpu.make_async_copy(src, dst, sem).wait()

    for j in range(4):
        fire_gather(jnp.int32(j), j)

    @pl.loop(0, _S // 4)
    def _k(k):
        for j in range(4):
            s = k * 4 + j
            drain(rows[j], gsems[j], tab.at[pl.ds(0, 128)])

            @pl.when(k > 0)
            def _():
                drain(out.at[0, :, 0], wsems[j], tiles[j])

            ssp = jnp.full((16,), s, jnp.int32)

            @pl.loop(0, _D)
            def _d(d):
                dsp = jnp.full((16,), d, jnp.int32)
                ps = plsc.load_gather(pos_v, [ssp, dsp])
                dq = d // 8
                dr = lax.rem(d, 8)
                for b in range(8):
                    v = plsc.load_gather(rows[j], [bcs[b], dsp]) + ps
                    tiles[j][dq, dr, pl.ds(16 * b, 16)] = v

            pltpu.async_copy(tiles[j], out.at[s, :, wid], wsems[j])

            @pl.when(k < _S // 4 - 1)
            def _():
                fire_gather(s + 4, j)

    for j in range(4):
        drain(out.at[0, :, 0], wsems[j], tiles[j])


@jax.jit
def _emb(x, token_table, pos_table):
    mesh = plsc.VectorSubcoreMesh(
        core_axis_name="c", subcore_axis_name="s", num_cores=2, num_subcores=16
    )
    cp = pltpu.CompilerParams(
        use_tc_tiling_on_sc=False, needs_layout_passes=False
    )

    f_tr = pl.kernel(
        _tr_body,
        out_type=jax.ShapeDtypeStruct((_V, _D), jnp.float32),
        mesh=mesh,
        scratch_types=[
            pltpu.VMEM((_D, _TC), jnp.float32),
            pltpu.VMEM((_D, _TC), jnp.float32),
            pltpu.VMEM((_TC, _D), jnp.float32),
            pltpu.VMEM((_TC, _D), jnp.float32),
            pltpu.SemaphoreType.DMA,
            pltpu.SemaphoreType.DMA,
            pltpu.SemaphoreType.DMA,
            pltpu.SemaphoreType.DMA,
        ],
        compiler_params=cp,
    )

    f_lk = pl.kernel(
        _lk_body,
        out_type=jax.ShapeDtypeStruct((_S, 4, 32, 8, 128), jnp.float32),
        mesh=mesh,
        scratch_types=(
            [
                pltpu.VMEM((25, 8, 128), jnp.int32),
                pltpu.VMEM((_S, _D), jnp.float32),
            ]
            + [pltpu.VMEM((128, _D), jnp.float32)] * 4
            + [pltpu.VMEM((4, 8, 128), jnp.float32)] * 4
            + [pltpu.SemaphoreType.DMA] * 8
        ),
        compiler_params=cp,
    )

    xv = x.T.reshape(25, 8, 32, 128).transpose(0, 2, 1, 3)
    tab = f_tr(token_table.T)
    V = f_lk(xv, tab, pos_table)
    return V.transpose(2, 4, 0, 1, 3).reshape(_B, _S, _D)


def kernel(x, token_table, pos_table):
    return _emb(x, token_table, pos_table)


# speedup vs baseline: 3.2500x; 3.2500x over previous
import functools

import jax
import jax.numpy as jnp
from jax import lax
from jax.experimental import pallas as pl
from jax.experimental.pallas import tpu as pltpu
from jax.experimental.pallas import tpu_sc as plsc

_B = 4096
_S = 200
_D = 32
_V = 1000000
_NW = 32


def _lk_body(
    xv, tab, pos_hbm, out,
    xidx, pos_v,
    r0, r1, r2, r3, t0, t1, t2, t3,
    g0, g1, g2, g3, w0, w1, w2, w3,
):
    cid = lax.axis_index("c")
    sid = lax.axis_index("s")
    wid = sid * 2 + cid

    rows = (r0, r1, r2, r3)
    tiles = (t0, t1, t2, t3)
    gsems = (g0, g1, g2, g3)
    wsems = (w0, w1, w2, w3)

    pltpu.sync_copy(pos_hbm, pos_v)
    pltpu.sync_copy(xv.at[:, wid], xidx)

    iota = lax.iota(jnp.int32, 16)
    bcs = [iota + 16 * b for b in range(8)]

    def fire_gather(s, j):
        pltpu.async_copy(
            tab.at[xidx.at[s // 8, lax.rem(s, 8)]], rows[j], gsems[j]
        )

    def drain(dst, sem, src):
        pltpu.make_async_copy(src, dst, sem).wait()

    for j in range(4):
        fire_gather(jnp.int32(j), j)

    @pl.loop(0, _S // 4)
    def _k(k):
        for j in range(4):
            s = k * 4 + j
            drain(rows[j], gsems[j], tab.at[pl.ds(0, 128)])

            @pl.when(k > 0)
            def _():
                drain(out.at[0, :, 0], wsems[j], tiles[j])

            ssp = jnp.full((16,), s, jnp.int32)

            @pl.loop(0, _D, unroll=8)
            def _d(d):
                dsp = jnp.full((16,), d, jnp.int32)
                ps = plsc.load_gather(pos_v, [ssp, dsp])
                dq = d // 8
                dr = lax.rem(d, 8)
                for b in range(8):
                    v = plsc.load_gather(rows[j], [bcs[b], dsp]) + ps
                    tiles[j][dq, dr, pl.ds(16 * b, 16)] = v

            pltpu.async_copy(tiles[j], out.at[s, :, wid], wsems[j])

            @pl.when(k < _S // 4 - 1)
            def _():
                fire_gather(s + 4, j)

    for j in range(4):
        drain(out.at[0, :, 0], wsems[j], tiles[j])


@jax.jit
def _emb(x, token_table, pos_table):
    mesh = plsc.VectorSubcoreMesh(
        core_axis_name="c", subcore_axis_name="s", num_cores=2, num_subcores=16
    )
    cp = pltpu.CompilerParams(
        use_tc_tiling_on_sc=False, needs_layout_passes=False
    )

    f_lk = pl.kernel(
        _lk_body,
        out_type=jax.ShapeDtypeStruct((_S, 4, 32, 8, 128), jnp.float32),
        mesh=mesh,
        scratch_types=(
            [
                pltpu.VMEM((25, 8, 128), jnp.int32),
                pltpu.VMEM((_S, _D), jnp.float32),
            ]
            + [pltpu.VMEM((128, _D), jnp.float32)] * 4
            + [pltpu.VMEM((4, 8, 128), jnp.float32)] * 4
            + [pltpu.SemaphoreType.DMA] * 8
        ),
        compiler_params=cp,
    )

    xv = x.T.reshape(25, 8, 32, 128).transpose(0, 2, 1, 3)
    V = f_lk(xv, token_table, pos_table)
    return V.transpose(2, 4, 0, 1, 3).reshape(_B, _S, _D)


def kernel(x, token_table, pos_table):
    return _emb(x, token_table, pos_table)


# speedup vs baseline: 4.3516x; 1.3390x over previous
import functools

import jax
import jax.numpy as jnp
from jax import lax
from jax.experimental import pallas as pl
from jax.experimental.pallas import tpu as pltpu
from jax.experimental.pallas import tpu_sc as plsc

_B = 4096
_S = 200
_D = 32
_V = 1000000
_NW = 32


def _lk_body(
    xv, tab, pos_hbm, out,
    xidx, pos_v,
    r0, r1, r2, r3, t0, t1, t2, t3,
    g0, g1, g2, g3, w0, w1, w2, w3,
):
    cid = lax.axis_index("c")
    sid = lax.axis_index("s")
    wid = sid * 2 + cid

    rows = (r0, r1, r2, r3)
    tiles = (t0, t1, t2, t3)
    gsems = (g0, g1, g2, g3)
    wsems = (w0, w1, w2, w3)

    pltpu.sync_copy(pos_hbm, pos_v)
    pltpu.sync_copy(xv.at[:, wid], xidx)

    iota = lax.iota(jnp.int32, 16)
    bcs = [iota + 16 * b for b in range(8)]

    def fire_gather(s, j):
        pltpu.async_copy(
            tab.at[xidx.at[s // 8, lax.rem(s, 8)]], rows[j], gsems[j]
        )

    def drain(dst, sem, src):
        pltpu.make_async_copy(src, dst, sem).wait()

    for j in range(4):
        fire_gather(jnp.int32(j), j)

    @pl.loop(0, _S // 4)
    def _k(k):
        for j in range(4):
            s = k * 4 + j
            drain(rows[j], gsems[j], tab.at[pl.ds(0, 128)])

            @pl.when(k > 0)
            def _():
                drain(out.at[0, :, 0], wsems[j], tiles[j])

            ssp = jnp.full((16,), s, jnp.int32)

            @plsc.parallel_loop(0, _D, unroll=8)
            def _d(d):
                dsp = jnp.full((16,), d, jnp.int32)
                ps = plsc.load_gather(pos_v, [ssp, dsp])
                dq = d // 8
                dr = lax.rem(d, 8)
                for b in range(8):
                    v = plsc.load_gather(rows[j], [bcs[b], dsp]) + ps
                    tiles[j][dq, dr, pl.ds(16 * b, 16)] = v

            pltpu.async_copy(tiles[j], out.at[s, :, wid], wsems[j])

            @pl.when(k < _S // 4 - 1)
            def _():
                fire_gather(s + 4, j)

    for j in range(4):
        drain(out.at[0, :, 0], wsems[j], tiles[j])


@jax.jit
def _emb(x, token_table, pos_table):
    mesh = plsc.VectorSubcoreMesh(
        core_axis_name="c", subcore_axis_name="s", num_cores=2, num_subcores=16
    )
    cp = pltpu.CompilerParams(
        use_tc_tiling_on_sc=False, needs_layout_passes=False
    )

    f_lk = pl.kernel(
        _lk_body,
        out_type=jax.ShapeDtypeStruct((_S, 4, 32, 8, 128), jnp.float32),
        mesh=mesh,
        scratch_types=(
            [
                pltpu.VMEM((25, 8, 128), jnp.int32),
                pltpu.VMEM((_S, _D), jnp.float32),
            ]
            + [pltpu.VMEM((128, _D), jnp.float32)] * 4
            + [pltpu.VMEM((4, 8, 128), jnp.float32)] * 4
            + [pltpu.SemaphoreType.DMA] * 8
        ),
        compiler_params=cp,
    )

    xv = x.T.reshape(25, 8, 32, 128).transpose(0, 2, 1, 3)
    V = f_lk(xv, token_table, pos_table)
    return V.transpose(2, 4, 0, 1, 3).reshape(_B, _S, _D)


def kernel(x, token_table, pos_table):
    return _emb(x, token_table, pos_table)


# speedup vs baseline: 4.4305x; 1.0181x over previous
import functools

import jax
import jax.numpy as jnp
from jax import lax
from jax.experimental import pallas as pl
from jax.experimental.pallas import tpu as pltpu
from jax.experimental.pallas import tpu_sc as plsc

_B = 4096
_S = 200
_D = 32
_V = 1000000
_NW = 32


def _lk_body(
    xv, tab, pos_hbm, out,
    xidx, pos_v,
    r0, r1, r2, r3, t0, t1, t2, t3,
    g0, g1, g2, g3, w0, w1, w2, w3,
):
    cid = lax.axis_index("c")
    sid = lax.axis_index("s")
    wid = sid * 2 + cid

    rows = (r0, r1, r2, r3)
    tiles = (t0, t1, t2, t3)
    gsems = (g0, g1, g2, g3)
    wsems = (w0, w1, w2, w3)

    pltpu.sync_copy(pos_hbm, pos_v)
    pltpu.sync_copy(xv.at[:, wid], xidx)

    iota = lax.iota(jnp.int32, 16)
    bcs = [iota + 16 * b for b in range(8)]

    def fire_gather(s, j):
        pltpu.async_copy(
            tab.at[xidx.at[s // 8, lax.rem(s, 8)]], rows[j], gsems[j]
        )

    def drain(dst, sem, src):
        pltpu.make_async_copy(src, dst, sem).wait()

    for j in range(4):
        fire_gather(jnp.int32(j), j)

    @pl.loop(0, _S // 4)
    def _k(k):
        for j in range(4):
            s = k * 4 + j
            drain(rows[j], gsems[j], tab.at[pl.ds(0, 128)])

            @pl.when(k > 0)
            def _():
                drain(out.at[0, :, 0], wsems[j], tiles[j])

            ssp = jnp.full((16,), s, jnp.int32)

            @plsc.parallel_loop(0, _D, unroll=16)
            def _d(d):
                dsp = jnp.full((16,), d, jnp.int32)
                ps = plsc.load_gather(pos_v, [ssp, dsp])
                dq = d // 8
                dr = lax.rem(d, 8)
                for b in range(8):
                    v = plsc.load_gather(rows[j], [bcs[b], dsp]) + ps
                    tiles[j][dq, dr, pl.ds(16 * b, 16)] = v

            pltpu.async_copy(tiles[j], out.at[s, :, wid], wsems[j])

            @pl.when(k < _S // 4 - 1)
            def _():
                fire_gather(s + 4, j)

    for j in range(4):
        drain(out.at[0, :, 0], wsems[j], tiles[j])


@jax.jit
def _emb(x, token_table, pos_table):
    mesh = plsc.VectorSubcoreMesh(
        core_axis_name="c", subcore_axis_name="s", num_cores=2, num_subcores=16
    )
    cp = pltpu.CompilerParams(
        use_tc_tiling_on_sc=False, needs_layout_passes=False
    )

    f_lk = pl.kernel(
        _lk_body,
        out_type=jax.ShapeDtypeStruct((_S, 4, 32, 8, 128), jnp.float32),
        mesh=mesh,
        scratch_types=(
            [
                pltpu.VMEM((25, 8, 128), jnp.int32),
                pltpu.VMEM((_S, _D), jnp.float32),
            ]
            + [pltpu.VMEM((128, _D), jnp.float32)] * 4
            + [pltpu.VMEM((4, 8, 128), jnp.float32)] * 4
            + [pltpu.SemaphoreType.DMA] * 8
        ),
        compiler_params=cp,
    )

    xv = x.T.reshape(25, 8, 32, 128).transpose(0, 2, 1, 3)
    V = f_lk(xv, token_table, pos_table)
    return V.transpose(2, 4, 0, 1, 3).reshape(_B, _S, _D)


def kernel(x, token_table, pos_table):
    return _emb(x, token_table, pos_table)
